# baseline (speedup 1.0000x reference)
import functools

import jax
import jax.numpy as jnp
from jax import lax
from jax.experimental import pallas as pl
from jax.experimental.pallas import tpu as pltpu
from jax.experimental.pallas import tpu_sc as plsc

SEQ = 8192
DIM = 128
CHUNK = 128


def _make_gather():
    info = plsc.get_sparse_core_info()
    num_workers = info.num_cores * info.num_subcores
    rows_per_worker = SEQ // num_workers
    n_chunks = rows_per_worker // CHUNK

    mesh = plsc.VectorSubcoreMesh(core_axis_name="c", subcore_axis_name="s")

    @functools.partial(
        pl.kernel,
        mesh=mesh,
        out_type=jax.ShapeDtypeStruct((SEQ, DIM), jnp.float32),
        scratch_types=[
            pltpu.VMEM((n_chunks, CHUNK), jnp.int32),
            pltpu.VMEM((n_chunks, CHUNK, DIM), jnp.float32),
        ]
        + [pltpu.SemaphoreType.DMA] * (n_chunks + 1),
    )
    def gather_kernel(idx_hbm, table_hbm, out_hbm, idx_v, rows_v, *sems):
        wid = lax.axis_index("s") * info.num_cores + lax.axis_index("c")
        base = wid * rows_per_worker
        for j in range(n_chunks):
            pltpu.sync_copy(
                idx_hbm.at[pl.ds(base + j * CHUNK, CHUNK)], idx_v.at[j]
            )
        gathers = [
            pltpu.async_copy(table_hbm.at[idx_v.at[j]], rows_v.at[j], sems[j])
            for j in range(n_chunks)
        ]
        writes = []
        for j in range(n_chunks):
            gathers[j].wait()
            writes.append(
                pltpu.async_copy(
                    rows_v.at[j],
                    out_hbm.at[pl.ds(base + j * CHUNK, CHUNK)],
                    sems[n_chunks],
                )
            )
        for w in writes:
            w.wait()

    return gather_kernel


_gather = _make_gather()


@jax.jit
def kernel(input_ids, table):
    ids = input_ids.astype(jnp.int32).reshape(SEQ)
    out = _gather(ids, table)
    return out.reshape(1, SEQ, DIM)


# speedup vs baseline: 1.0109x; 1.0109x over previous
import functools

import jax
import jax.numpy as jnp
from jax import lax
from jax.experimental import pallas as pl
from jax.experimental.pallas import tpu as pltpu
from jax.experimental.pallas import tpu_sc as plsc

SEQ = 8192
DIM = 128
CHUNK = 64


def _make_gather():
    info = plsc.get_sparse_core_info()
    num_workers = info.num_cores * info.num_subcores
    rows_per_worker = SEQ // num_workers
    n_chunks = rows_per_worker // CHUNK

    mesh = plsc.VectorSubcoreMesh(core_axis_name="c", subcore_axis_name="s")

    @functools.partial(
        pl.kernel,
        mesh=mesh,
        out_type=jax.ShapeDtypeStruct((SEQ, DIM), jnp.float32),
        scratch_types=[
            pltpu.VMEM((rows_per_worker,), jnp.int32),
            pltpu.VMEM((n_chunks, CHUNK, DIM), jnp.float32),
        ]
        + [pltpu.SemaphoreType.DMA] * (2 * n_chunks),
    )
    def gather_kernel(idx_hbm, table_hbm, out_hbm, idx_v, rows_v, *sems):
        wid = lax.axis_index("s") * info.num_cores + lax.axis_index("c")
        base = wid * rows_per_worker
        pltpu.sync_copy(idx_hbm.at[pl.ds(base, rows_per_worker)], idx_v)
        gathers = [
            pltpu.async_copy(
                table_hbm.at[idx_v.at[pl.ds(j * CHUNK, CHUNK)]],
                rows_v.at[j],
                sems[j],
            )
            for j in range(n_chunks)
        ]
        writes = []
        for j in range(n_chunks):
            gathers[j].wait()
            writes.append(
                pltpu.async_copy(
                    rows_v.at[j],
                    out_hbm.at[pl.ds(base + j * CHUNK, CHUNK)],
                    sems[n_chunks + j],
                )
            )
        for w in writes:
            w.wait()

    return gather_kernel


_gather = _make_gather()


@jax.jit
def kernel(input_ids, table):
    ids = input_ids.astype(jnp.int32).reshape(SEQ)
    out = _gather(ids, table)
    return out.reshape(1, SEQ, DIM)


# speedup vs baseline: 1.0303x; 1.0191x over previous
import functools

import jax
import jax.numpy as jnp
from jax import lax
from jax.experimental import pallas as pl
from jax.experimental.pallas import tpu as pltpu
from jax.experimental.pallas import tpu_sc as plsc

SEQ = 8192
DIM = 128
CHUNKS = (128, 64, 32, 16, 16)


def _make_gather():
    info = plsc.get_sparse_core_info()
    num_workers = info.num_cores * info.num_subcores
    rows_per_worker = SEQ // num_workers
    assert sum(CHUNKS) == rows_per_worker
    n_chunks = len(CHUNKS)
    offs = [sum(CHUNKS[:j]) for j in range(n_chunks)]

    mesh = plsc.VectorSubcoreMesh(core_axis_name="c", subcore_axis_name="s")

    @functools.partial(
        pl.kernel,
        mesh=mesh,
        out_type=jax.ShapeDtypeStruct((SEQ, DIM), jnp.float32),
        scratch_types=[
            pltpu.VMEM((rows_per_worker,), jnp.int32),
            pltpu.VMEM((rows_per_worker, DIM), jnp.float32),
        ]
        + [pltpu.SemaphoreType.DMA] * (n_chunks + 2),
    )
    def gather_kernel(idx_hbm, table_hbm, out_hbm, idx_v, rows_v, *sems):
        wid = lax.axis_index("s") * info.num_cores + lax.axis_index("c")
        base = wid * rows_per_worker
        c0 = CHUNKS[0]
        pltpu.sync_copy(idx_hbm.at[pl.ds(base, c0)], idx_v.at[pl.ds(0, c0)])
        gathers = [
            pltpu.async_copy(
                table_hbm.at[idx_v.at[pl.ds(0, c0)]],
                rows_v.at[pl.ds(0, c0)],
                sems[0],
            )
        ]
        rest = rows_per_worker - c0
        pltpu.sync_copy(
            idx_hbm.at[pl.ds(base + c0, rest)], idx_v.at[pl.ds(c0, rest)]
        )
        for j in range(1, n_chunks):
            gathers.append(
                pltpu.async_copy(
                    table_hbm.at[idx_v.at[pl.ds(offs[j], CHUNKS[j])]],
                    rows_v.at[pl.ds(offs[j], CHUNKS[j])],
                    sems[j],
                )
            )
        writes = []
        for j in range(n_chunks):
            gathers[j].wait()
            writes.append(
                pltpu.async_copy(
                    rows_v.at[pl.ds(offs[j], CHUNKS[j])],
                    out_hbm.at[pl.ds(base + offs[j], CHUNKS[j])],
                    sems[n_chunks],
                )
            )
        for w in writes:
            w.wait()

    return gather_kernel


_gather = _make_gather()


@jax.jit
def kernel(input_ids, table):
    ids = input_ids.astype(jnp.int32).reshape(SEQ)
    out = _gather(ids, table)
    return out.reshape(1, SEQ, DIM)
